# baseline (speedup 1.0000x reference)
import functools

import jax
import jax.numpy as jnp
from jax import lax
from jax.experimental import pallas as pl
from jax.experimental.pallas import tpu as pltpu, tpu_sc as plsc

N = 10000
E = 320000
D = 128
NW = 32
EPW = E // NW
CHUNK = 112
NCHUNK = 90
EPWP = NCHUNK * CHUNK
NPAD = 10240
TRASH = N
RPT = NPAD // 16
OCH = 80


def _sc_mesh():
    return plsc.VectorSubcoreMesh(core_axis_name="c", subcore_axis_name="s")


def _sc_degree(dst):
    @functools.partial(
        pl.kernel,
        out_type=jax.ShapeDtypeStruct((NW * N,), jnp.float32),
        mesh=_sc_mesh(),
        compiler_params=pltpu.CompilerParams(needs_layout_passes=False),
        scratch_types=[
            pltpu.VMEM((EPW,), jnp.int32),
            pltpu.VMEM((N,), jnp.float32),
        ],
    )
    def deg_kernel(dst_hbm, out_hbm, dbuf, degt):
        cid = lax.axis_index("c")
        sid = lax.axis_index("s")
        wid = cid * 16 + sid
        z16 = jnp.zeros((16,), jnp.float32)

        def zstep(i, carry):
            degt[pl.ds(i * 16, 16)] = z16
            return carry

        lax.fori_loop(0, N // 16, zstep, 0)
        pltpu.sync_copy(dst_hbm.at[pl.ds(wid * EPW, EPW)], dbuf)
        ones = jnp.ones((16,), jnp.float32)

        def step(i, carry):
            idx = dbuf[pl.ds(i * 16, 16)]
            plsc.addupdate_scatter(degt, [idx], ones)
            return carry

        lax.fori_loop(0, EPW // 16, step, 0)
        pltpu.sync_copy(degt, out_hbm.at[pl.ds(wid * N, N)])

    return deg_kernel(dst)


def _sc_spmm(y, srcw, dstw, rowidx):
    @functools.partial(
        pl.kernel,
        out_type=jax.ShapeDtypeStruct((2 * NPAD, D), jnp.float32),
        mesh=_sc_mesh(),
        compiler_params=pltpu.CompilerParams(needs_layout_passes=False),
        scratch_types=[
            [pltpu.VMEM((CHUNK,), jnp.int32)] * 6,
            [pltpu.VMEM((CHUNK,), jnp.int32)] * 6,
            [pltpu.VMEM((CHUNK, D), jnp.float32)] * 3,
            pltpu.VMEM((OCH,), jnp.int32),
            pltpu.VMEM_SHARED((NPAD, D), jnp.float32),
            [pltpu.SemaphoreType.DMA] * 3,
            [pltpu.SemaphoreType.DMA] * 3,
            [pltpu.SemaphoreType.DMA] * 6,
        ],
    )
    def spmm_kernel(y_hbm, src_hbm, dst_hbm, ridx_hbm, out_hbm,
                    idxs, idxd, rows, oidx, accum, gsem, ssem, isem):
        cid = lax.axis_index("c")
        sid = lax.axis_index("s")
        wid = cid * 16 + sid

        z16 = jnp.zeros((16,), jnp.float32)

        def zstep(i, carry):
            rows[0][i // 8, pl.ds((i % 8) * 16, 16)] = z16
            return carry

        lax.fori_loop(0, OCH * D // 16, zstep, 0)
        for j in range(RPT // OCH):
            pltpu.sync_copy(rows[0].at[pl.ds(0, OCH)],
                            accum.at[pl.ds(sid * RPT + j * OCH, OCH)])
        plsc.subcore_barrier()

        ebase = wid * EPWP

        def stage(c, q):
            b = ebase + c * CHUNK
            pltpu.async_copy(src_hbm.at[pl.ds(b, CHUNK)], idxs[q], isem[q])
            pltpu.async_copy(dst_hbm.at[pl.ds(b, CHUNK)], idxd[q], isem[q])

        def wait_stage(c, q):
            b = ebase + c * CHUNK
            pltpu.make_async_copy(src_hbm.at[pl.ds(b, CHUNK)], idxs[q],
                                  isem[q]).wait()
            pltpu.make_async_copy(dst_hbm.at[pl.ds(b, CHUNK)], idxd[q],
                                  isem[q]).wait()

        def fireg(p, q):
            pltpu.async_copy(y_hbm.at[idxs[q]], rows[p], gsem[p])

        def drain(p, q):
            pltpu.make_async_copy(y_hbm.at[idxs[q]], rows[p], gsem[p]).wait()
            pltpu.async_copy(rows[p], accum.at[idxd[q]], ssem[p], add=True)

        def wait_scatter(p, q):
            pltpu.make_async_copy(rows[p], accum.at[idxd[q]], ssem[p]).wait()

        stage(0, 0)
        stage(1, 1)
        stage(2, 2)
        wait_stage(0, 0)
        fireg(0, 0)
        wait_stage(1, 1)
        fireg(1, 1)

        def step(k, carry):
            for j in range(6):
                c = 6 * k + j
                p = j % 3
                q = j
                drain(p, q)
                if j == 0:
                    @pl.when(k > 0)
                    def _():
                        wait_scatter((j + 2) % 3, (j + 5) % 6)
                else:
                    wait_scatter((j + 2) % 3, (j + 5) % 6)
                wait_stage(c + 2, (j + 2) % 6)
                fireg((j + 2) % 3, (j + 2) % 6)
                stage(c + 3, (j + 3) % 6)
            return carry

        lax.fori_loop(0, NCHUNK // 6 - 1, step, 0)
        e = NCHUNK - 6
        drain(0, 0)
        wait_scatter(2, 5)
        wait_stage(e + 2, 2)
        fireg(2, 2)
        stage(e + 3, 3)
        drain(1, 1)
        wait_scatter(0, 0)
        wait_stage(e + 3, 3)
        fireg(0, 3)
        stage(e + 4, 4)
        drain(2, 2)
        wait_scatter(1, 1)
        wait_stage(e + 4, 4)
        fireg(1, 4)
        drain(0, 3)
        wait_scatter(2, 2)
        stage(e + 5, 5)
        wait_stage(e + 5, 5)
        fireg(2, 5)
        drain(1, 4)
        wait_scatter(0, 3)
        drain(2, 5)
        wait_scatter(1, 4)
        wait_scatter(2, 5)
        plsc.subcore_barrier()

        for j in range(RPT // OCH):
            start = sid * RPT + j * OCH
            pltpu.sync_copy(accum.at[pl.ds(start, OCH)],
                            rows[0].at[pl.ds(0, OCH)])
            pltpu.sync_copy(ridx_hbm.at[pl.ds(cid * NPAD + start, OCH)], oidx)
            pltpu.sync_copy(rows[0].at[pl.ds(0, OCH)], out_hbm.at[oidx])

    return spmm_kernel(y, srcw, dstw, rowidx)


def _tc_mm(x, w):
    def body(x_ref, w_ref, o_ref):
        o_ref[...] = jnp.dot(x_ref[...], w_ref[...],
                             preferred_element_type=jnp.float32,
                             precision=lax.Precision.HIGHEST)

    return pl.pallas_call(
        body,
        out_shape=jax.ShapeDtypeStruct((x.shape[0], w.shape[1]), jnp.float32),
    )(x, w)


def _tc_prep(degp, xw):
    def body(degp_ref, xw_ref, dis_ref, y_ref):
        deg = jnp.sum(degp_ref[...], axis=1, keepdims=True) + 1.0
        dis = lax.rsqrt(deg)
        dis_ref[...] = dis
        y_ref[...] = xw_ref[...] * dis

    return pl.pallas_call(
        body,
        out_shape=[
            jax.ShapeDtypeStruct((N, 1), jnp.float32),
            jax.ShapeDtypeStruct(xw.shape, jnp.float32),
        ],
    )(degp, xw)


def _tc_stage(P, y, dis, b, g, bt, w_next):
    def body(p_ref, y_ref, dis_ref, b_ref, g_ref, bt_ref, w_ref,
             yn_ref):
        dis = dis_ref[...]
        s = p_ref[...]
        c = dis * (s[:N] + s[NPAD:NPAD + N] + y_ref[...]) + b_ref[...]
        mu = jnp.mean(c, axis=0, keepdims=True)
        var = jnp.mean((c - mu) ** 2, axis=0, keepdims=True)
        h = (c - mu) * lax.rsqrt(var + 1e-5) * g_ref[...] + bt_ref[...]
        h = jnp.maximum(h, 0.0)
        yn_ref[...] = dis * jnp.dot(h, w_ref[...],
                                    preferred_element_type=jnp.float32,
                                    precision=lax.Precision.HIGHEST)

    return pl.pallas_call(
        body,
        out_shape=jax.ShapeDtypeStruct((N, w_next.shape[1]), jnp.float32),
    )(P, y, dis, b, g, bt, w_next)


def _tc_stage_nomm(P, y, dis, b, g, bt):
    def body(p_ref, y_ref, dis_ref, b_ref, g_ref, bt_ref, yn_ref):
        dis = dis_ref[...]
        s = p_ref[...]
        c = dis * (s[:N] + s[NPAD:NPAD + N] + y_ref[...]) + b_ref[...]
        mu = jnp.mean(c, axis=0, keepdims=True)
        var = jnp.mean((c - mu) ** 2, axis=0, keepdims=True)
        h = (c - mu) * lax.rsqrt(var + 1e-5) * g_ref[...] + bt_ref[...]
        yn_ref[...] = dis * jnp.maximum(h, 0.0)

    return pl.pallas_call(
        body,
        out_shape=jax.ShapeDtypeStruct((N, D), jnp.float32),
    )(P, y, dis, b, g, bt)


def _tc_final(P, y, dis, w, b):
    def body(p_ref, y_ref, dis_ref, w_ref, b_ref, o_ref):
        s = p_ref[...]
        t = s[:N] + s[NPAD:NPAD + N] + y_ref[...]
        logits = dis_ref[...] * jnp.dot(t, w_ref[...],
                                        preferred_element_type=jnp.float32,
                                        precision=lax.Precision.HIGHEST)
        logits = logits + b_ref[...]
        m = jnp.max(logits, axis=1, keepdims=True)
        lse = m + jnp.log(jnp.sum(jnp.exp(logits - m), axis=1, keepdims=True))
        o_ref[...] = logits - lse

    return pl.pallas_call(
        body,
        out_shape=jax.ShapeDtypeStruct((N, w.shape[1]), jnp.float32),
    )(P, y, dis, w, b)


def kernel(x, edge_index, W1, b1, g1, bt1, W2, b2, g2, bt2, W3, b3):
    src = edge_index[0].astype(jnp.int32)
    dst = edge_index[1].astype(jnp.int32)
    rowidx = jnp.arange(2 * NPAD, dtype=jnp.int32)

    pad = EPWP - EPW
    srcw = jnp.concatenate(
        [src.reshape(NW, EPW), jnp.zeros((NW, pad), jnp.int32)],
        axis=1).reshape(NW * EPWP)
    dstw = jnp.concatenate(
        [dst.reshape(NW, EPW), jnp.full((NW, pad), TRASH, jnp.int32)],
        axis=1).reshape(NW * EPWP)

    degp = _sc_degree(dst).reshape(NW, N).T
    xw1 = _tc_mm(x, W1)
    dis, y1 = _tc_prep(degp, xw1)

    P = _sc_spmm(y1, srcw, dstw, rowidx)
    y2 = _tc_stage(P, y1, dis, b1.reshape(1, -1),
                   g1.reshape(1, -1), bt1.reshape(1, -1), W2)

    P = _sc_spmm(y2, srcw, dstw, rowidx)
    y3 = _tc_stage_nomm(P, y2, dis, b2.reshape(1, -1),
                        g2.reshape(1, -1), bt2.reshape(1, -1))

    P = _sc_spmm(y3, srcw, dstw, rowidx)
    return _tc_final(P, y3, dis, W3, b3.reshape(1, -1))


# speedup vs baseline: 1.6740x; 1.6740x over previous
import functools

import jax
import jax.numpy as jnp
from jax import lax
from jax.experimental import pallas as pl
from jax.experimental.pallas import tpu as pltpu, tpu_sc as plsc

N = 10000
E = 320000
D = 128
NW = 32
EPW = E // NW
CHUNK = 80
NCHUNK = EPW // CHUNK
NPAD = 10240
RPT = NPAD // 16
OCH = 128


def _sc_mesh():
    return plsc.VectorSubcoreMesh(core_axis_name="c", subcore_axis_name="s")


def _sc_degree(dst):
    @functools.partial(
        pl.kernel,
        out_type=jax.ShapeDtypeStruct((NW * N,), jnp.float32),
        mesh=_sc_mesh(),
        compiler_params=pltpu.CompilerParams(needs_layout_passes=False),
        scratch_types=[
            pltpu.VMEM((EPW,), jnp.int32),
            pltpu.VMEM((N,), jnp.float32),
        ],
    )
    def deg_kernel(dst_hbm, out_hbm, dbuf, degt):
        cid = lax.axis_index("c")
        sid = lax.axis_index("s")
        wid = cid * 16 + sid
        z16 = jnp.zeros((16,), jnp.float32)

        def zstep(i, carry):
            degt[pl.ds(i * 16, 16)] = z16
            return carry

        lax.fori_loop(0, N // 16, zstep, 0)
        pltpu.sync_copy(dst_hbm.at[pl.ds(wid * EPW, EPW)], dbuf)
        ones = jnp.ones((16,), jnp.float32)

        def step(i, carry):
            idx = dbuf[pl.ds(i * 16, 16)]
            plsc.addupdate_scatter(degt, [idx], ones)
            return carry

        lax.fori_loop(0, EPW // 16, step, 0)
        pltpu.sync_copy(degt, out_hbm.at[pl.ds(wid * N, N)])

    return deg_kernel(dst)


def _sc_spmm(y, srcw, dstw, rowidx):
    @functools.partial(
        pl.kernel,
        out_type=jax.ShapeDtypeStruct((2 * NPAD, D), jnp.float32),
        mesh=_sc_mesh(),
        compiler_params=pltpu.CompilerParams(needs_layout_passes=False),
        scratch_types=[
            [pltpu.VMEM((CHUNK,), jnp.int32)] * 6,
            [pltpu.VMEM((CHUNK,), jnp.int32)] * 6,
            [pltpu.VMEM((CHUNK, D), jnp.float32)] * 3,
            pltpu.VMEM((OCH, D), jnp.float32),
            pltpu.VMEM((OCH,), jnp.int32),
            pltpu.VMEM_SHARED((NPAD, D), jnp.float32),
            [pltpu.SemaphoreType.DMA] * 3,
            [pltpu.SemaphoreType.DMA] * 3,
            [pltpu.SemaphoreType.DMA] * 6,
        ],
    )
    def spmm_kernel(y_hbm, src_hbm, dst_hbm, ridx_hbm, out_hbm,
                    idxs, idxd, rows, slab, oidx, accum, gsem, ssem, isem):
        cid = lax.axis_index("c")
        sid = lax.axis_index("s")
        wid = cid * 16 + sid

        z16 = jnp.zeros((16,), jnp.float32)

        def zstep(i, carry):
            slab[i // 8, pl.ds((i % 8) * 16, 16)] = z16
            return carry

        lax.fori_loop(0, OCH * D // 16, zstep, 0)
        for j in range(RPT // OCH):
            pltpu.sync_copy(slab, accum.at[pl.ds(sid * RPT + j * OCH, OCH)])
        plsc.subcore_barrier()

        ebase = wid * EPW

        def stage(c, q):
            b = ebase + c * CHUNK
            pltpu.async_copy(src_hbm.at[pl.ds(b, CHUNK)], idxs[q], isem[q])
            pltpu.async_copy(dst_hbm.at[pl.ds(b, CHUNK)], idxd[q], isem[q])

        def wait_stage(c, q):
            b = ebase + c * CHUNK
            pltpu.make_async_copy(src_hbm.at[pl.ds(b, CHUNK)], idxs[q],
                                  isem[q]).wait()
            pltpu.make_async_copy(dst_hbm.at[pl.ds(b, CHUNK)], idxd[q],
                                  isem[q]).wait()

        def fireg(p, q):
            pltpu.async_copy(y_hbm.at[idxs[q]], rows[p], gsem[p])

        def drain(p, q):
            pltpu.make_async_copy(y_hbm.at[idxs[q]], rows[p], gsem[p]).wait()
            pltpu.async_copy(rows[p], accum.at[idxd[q]], ssem[p], add=True)

        def wait_scatter(p, q):
            pltpu.make_async_copy(rows[p], accum.at[idxd[q]], ssem[p]).wait()

        stage(0, 0)
        stage(1, 1)
        stage(2, 2)
        wait_stage(0, 0)
        fireg(0, 0)
        wait_stage(1, 1)
        fireg(1, 1)

        def step(k, carry):
            for j in range(6):
                c = 6 * k + j
                p = j % 3
                q = j
                drain(p, q)
                if j == 0:
                    @pl.when(k > 0)
                    def _():
                        wait_scatter((j + 2) % 3, (j + 5) % 6)
                else:
                    wait_scatter((j + 2) % 3, (j + 5) % 6)
                wait_stage(c + 2, (j + 2) % 6)
                fireg((j + 2) % 3, (j + 2) % 6)
                stage(c + 3, (j + 3) % 6)
            return carry

        lax.fori_loop(0, 20, step, 0)
        drain(0, 0)
        wait_scatter(2, 5)
        wait_stage(122, 2)
        fireg(2, 2)
        stage(123, 3)
        drain(1, 1)
        wait_scatter(0, 0)
        wait_stage(123, 3)
        fireg(0, 3)
        stage(124, 4)
        drain(2, 2)
        wait_scatter(1, 1)
        wait_stage(124, 4)
        fireg(1, 4)
        drain(0, 3)
        wait_scatter(2, 2)
        drain(1, 4)
        wait_scatter(0, 3)
        wait_scatter(1, 4)
        plsc.subcore_barrier()

        for j in range(RPT // OCH):
            start = sid * RPT + j * OCH
            pltpu.sync_copy(accum.at[pl.ds(start, OCH)], slab)
            pltpu.sync_copy(ridx_hbm.at[pl.ds(cid * NPAD + start, OCH)], oidx)
            pltpu.sync_copy(slab, out_hbm.at[oidx])

    return spmm_kernel(y, srcw, dstw, rowidx)


def _tc_mm(x, w):
    def body(x_ref, w_ref, o_ref):
        o_ref[...] = jnp.dot(x_ref[...], w_ref[...],
                             preferred_element_type=jnp.float32,
                             precision=lax.Precision.HIGHEST)

    return pl.pallas_call(
        body,
        out_shape=jax.ShapeDtypeStruct((x.shape[0], w.shape[1]), jnp.float32),
    )(x, w)


def _tc_prep(degp, xw):
    def body(degp_ref, xw_ref, dis_ref, y_ref):
        deg = jnp.sum(degp_ref[...], axis=1, keepdims=True) + 1.0
        dis = lax.rsqrt(deg)
        dis_ref[...] = dis
        y_ref[...] = xw_ref[...] * dis

    return pl.pallas_call(
        body,
        out_shape=[
            jax.ShapeDtypeStruct((N, 1), jnp.float32),
            jax.ShapeDtypeStruct(xw.shape, jnp.float32),
        ],
    )(degp, xw)


def _tc_stage(P, y, dis, b, g, bt, w_next):
    def body(p_ref, y_ref, dis_ref, b_ref, g_ref, bt_ref, w_ref,
             yn_ref):
        dis = dis_ref[...]
        s = p_ref[...]
        c = dis * (s[:N] + s[NPAD:NPAD + N] + y_ref[...]) + b_ref[...]
        mu = jnp.mean(c, axis=0, keepdims=True)
        var = jnp.mean((c - mu) ** 2, axis=0, keepdims=True)
        h = (c - mu) * lax.rsqrt(var + 1e-5) * g_ref[...] + bt_ref[...]
        h = jnp.maximum(h, 0.0)
        yn_ref[...] = dis * jnp.dot(h, w_ref[...],
                                    preferred_element_type=jnp.float32,
                                    precision=lax.Precision.HIGHEST)

    return pl.pallas_call(
        body,
        out_shape=jax.ShapeDtypeStruct((N, w_next.shape[1]), jnp.float32),
    )(P, y, dis, b, g, bt, w_next)


def _tc_stage_nomm(P, y, dis, b, g, bt):
    def body(p_ref, y_ref, dis_ref, b_ref, g_ref, bt_ref, yn_ref):
        dis = dis_ref[...]
        s = p_ref[...]
        c = dis * (s[:N] + s[NPAD:NPAD + N] + y_ref[...]) + b_ref[...]
        mu = jnp.mean(c, axis=0, keepdims=True)
        var = jnp.mean((c - mu) ** 2, axis=0, keepdims=True)
        h = (c - mu) * lax.rsqrt(var + 1e-5) * g_ref[...] + bt_ref[...]
        yn_ref[...] = dis * jnp.maximum(h, 0.0)

    return pl.pallas_call(
        body,
        out_shape=jax.ShapeDtypeStruct((N, D), jnp.float32),
    )(P, y, dis, b, g, bt)


def _tc_final(P, y, dis, w, b):
    def body(p_ref, y_ref, dis_ref, w_ref, b_ref, o_ref):
        s = p_ref[...]
        t = s[:N] + s[NPAD:NPAD + N] + y_ref[...]
        logits = dis_ref[...] * jnp.dot(t, w_ref[...],
                                        preferred_element_type=jnp.float32,
                                        precision=lax.Precision.HIGHEST)
        logits = logits + b_ref[...]
        m = jnp.max(logits, axis=1, keepdims=True)
        lse = m + jnp.log(jnp.sum(jnp.exp(logits - m), axis=1, keepdims=True))
        o_ref[...] = logits - lse

    return pl.pallas_call(
        body,
        out_shape=jax.ShapeDtypeStruct((N, w.shape[1]), jnp.float32),
    )(P, y, dis, w, b)


def kernel(x, edge_index, W1, b1, g1, bt1, W2, b2, g2, bt2, W3, b3):
    src = edge_index[0].astype(jnp.int32)
    dst = edge_index[1].astype(jnp.int32)
    rowidx = jnp.arange(2 * NPAD, dtype=jnp.int32)

    srcw, dstw = src, dst

    degp = _sc_degree(dst).reshape(NW, N).T
    xw1 = _tc_mm(x, W1)
    dis, y1 = _tc_prep(degp, xw1)

    P = _sc_spmm(y1, srcw, dstw, rowidx)
    y2 = _tc_stage(P, y1, dis, b1.reshape(1, -1),
                   g1.reshape(1, -1), bt1.reshape(1, -1), W2)

    P = _sc_spmm(y2, srcw, dstw, rowidx)
    y3 = _tc_stage_nomm(P, y2, dis, b2.reshape(1, -1),
                        g2.reshape(1, -1), bt2.reshape(1, -1))

    P = _sc_spmm(y3, srcw, dstw, rowidx)
    return _tc_final(P, y3, dis, W3, b3.reshape(1, -1))
